# baseline (speedup 1.0000x reference)
import functools

import jax
import jax.numpy as jnp
from jax import lax
from jax.experimental import pallas as pl
from jax.experimental.pallas import tpu as pltpu
from jax.experimental.pallas import tpu_sc as plsc

_BATCH = 16384
_DIM = 32
_VOCAB = 1000000
_FULL = 999936
_TAIL = _VOCAB - _FULL

_info = plsc.get_sparse_core_info()
_NC, _NS = _info.num_cores, _info.num_subcores
_NW = _NC * _NS

_LT = 128
_NLT = _FULL // _LT
_BASE_LT = _NLT // _NW
_EXTRA = _NLT - _BASE_LT * _NW

_CHUNK = 1280
_NFULL = (_BASE_LT * _LT) // _CHUNK
_REM0 = _BASE_LT * _LT - _NFULL * _CHUNK + _LT
_REM1 = _BASE_LT * _LT - _NFULL * _CHUNK

_LIST_CAP = 1024
_MATCH_CAP = 256
_SENT = 1 << 30


def _scalar(ref, j):
    return ref[pl.ds(j, 16)][0]


def _lookup(indices, tab_t, tab_tail):
    mesh = plsc.VectorSubcoreMesh(core_axis_name="c", subcore_axis_name="s")

    @functools.partial(
        pl.kernel,
        mesh=mesh,
        compiler_params=pltpu.CompilerParams(needs_layout_passes=False),
        out_type=jax.ShapeDtypeStruct((_BATCH * _DIM,), jnp.float32),
        scratch_types=[
            pltpu.VMEM((_BATCH,), jnp.int32),
            pltpu.VMEM((_LIST_CAP + 16,), jnp.int32),
            pltpu.VMEM((_LIST_CAP + 16,), jnp.int32),
            pltpu.VMEM((_MATCH_CAP + 16,), jnp.int32),
            pltpu.VMEM((_MATCH_CAP + 16,), jnp.int32),
            pltpu.VMEM((_DIM, _CHUNK), jnp.float32),
            pltpu.VMEM((_DIM, _CHUNK), jnp.float32),
            pltpu.VMEM((_TAIL, _DIM), jnp.float32),
            pltpu.VMEM((_MATCH_CAP * _DIM,), jnp.float32),
            pltpu.SemaphoreType.DMA,
            pltpu.SemaphoreType.DMA,
            pltpu.SemaphoreType.DMA,
        ],
    )
    def k(idx_hbm, tab_hbm, tail_hbm, out_hbm, idx_all, vals, bpos, mcol, mb,
          cbuf0, cbuf1, tail_v, stage, csem0, csem1, rsem):
        w = lax.axis_index("s") * _NC + lax.axis_index("c")
        lo_lane = _LT * (_BASE_LT * w + jnp.minimum(w, _EXTRA))
        n_lt = _BASE_LT + jnp.where(w < _EXTRA, 1, 0)
        hi = jnp.where(w == _NW - 1, _VOCAB, lo_lane + n_lt * _LT)

        cbufs = (cbuf0, cbuf1)
        csems = (csem0, csem1)
        iota = lax.iota(jnp.int32, 16)

        def chunk_lo(c):
            return pl.multiple_of(lo_lane + c * _CHUNK, _LT)

        def start_chunk(c, size):
            return pltpu.async_copy(
                tab_hbm.at[:, pl.ds(chunk_lo(c), size)],
                cbufs[c % 2].at[:, pl.ds(0, size)],
                csems[c % 2],
            )

        cp0 = start_chunk(0, _CHUNK)
        cp1 = start_chunk(1, _CHUNK)

        pltpu.sync_copy(idx_hbm, idx_all)

        def filt(t, off):
            for u in range(4):
                vec = idx_all[pl.ds((t * 4 + u) * 16, 16)]
                m = (vec >= lo_lane) & (vec < hi)
                cnt = plsc.all_reduce_population_count(m)[0]
                off = jnp.minimum(off, _LIST_CAP - 16)
                plsc.store_compressed(vals.at[pl.ds(off, 16)], vec, mask=m)
                b = iota + (t * 4 + u) * 16
                plsc.store_compressed(bpos.at[pl.ds(off, 16)], b, mask=m)
                off = off + cnt
            return off

        total = lax.fori_loop(0, _BATCH // 64, filt, jnp.int32(0))
        vals[pl.ds(total, 16)] = jnp.full((16,), _SENT, jnp.int32)
        n_vec = (total + 15) // 16

        def scan_matches(lo_c, hi_c):
            def scan(t, off2):
                vec = vals[pl.ds(t * 16, 16)]
                m = (vec >= lo_c) & (vec < hi_c)
                cnt = plsc.all_reduce_population_count(m)[0]
                off2 = jnp.minimum(off2, _MATCH_CAP - 16)
                plsc.store_compressed(mcol.at[pl.ds(off2, 16)], vec - lo_c, mask=m)
                bvec = bpos[pl.ds(t * 16, 16)]
                plsc.store_compressed(mb.at[pl.ds(off2, 16)], bvec, mask=m)
                return off2 + cnt
            return lax.fori_loop(0, n_vec, scan, jnp.int32(0))

        def emit_rows(m_total, buf):
            def emit16(g, _):
                colv = jnp.clip(mcol[pl.ds(g * 16, 16)], 0, _CHUNK - 1)
                sidx = iota * _DIM + g * (16 * _DIM)

                def per_dim(d, _):
                    dv = jnp.full((16,), d, jnp.int32)
                    gd = plsc.load_gather(buf, [dv, colv])
                    plsc.store_scatter(stage, [sidx + d], gd)
                    return 0

                lax.fori_loop(0, _DIM, per_dim, 0)
                return 0
            lax.fori_loop(0, (m_total + 15) // 16, emit16, 0)

            def send(j, _):
                b = _scalar(mb, j)
                pltpu.async_copy(
                    stage.at[pl.ds(j * _DIM, _DIM)],
                    out_hbm.at[pl.ds(b * _DIM, _DIM)],
                    rsem,
                )
                return 0
            lax.fori_loop(0, m_total, send, 0)

            def drain(j, _):
                pltpu.make_async_copy(
                    out_hbm.at[pl.ds(0, _DIM)],
                    stage.at[pl.ds(0, _DIM)],
                    rsem,
                ).wait()
                return 0
            lax.fori_loop(0, m_total, drain, 0)

        cps = [cp0, cp1]
        for c in range(_NFULL):
            cps[c % 2].wait()
            lo_c = chunk_lo(c)
            m_total = scan_matches(lo_c, lo_c + _CHUNK)
            emit_rows(m_total, cbufs[c % 2])
            if c + 2 < _NFULL:
                cps[c % 2] = start_chunk(c + 2, _CHUNK)
            elif c + 2 == _NFULL:
                @pl.when(w < _EXTRA)
                def _():
                    pltpu.async_copy(
                        tab_hbm.at[:, pl.ds(chunk_lo(_NFULL), _REM0)],
                        cbufs[_NFULL % 2].at[:, pl.ds(0, _REM0)],
                        csems[_NFULL % 2],
                    )

                @pl.when(w >= _EXTRA)
                def _():
                    pltpu.async_copy(
                        tab_hbm.at[:, pl.ds(chunk_lo(_NFULL), _REM1)],
                        cbufs[_NFULL % 2].at[:, pl.ds(0, _REM1)],
                        csems[_NFULL % 2],
                    )

        rem = jnp.where(w < _EXTRA, _REM0, _REM1)
        pltpu.make_async_copy(
            tab_hbm.at[:, pl.ds(0, _REM1)],
            cbufs[_NFULL % 2].at[:, pl.ds(0, _REM1)],
            csems[_NFULL % 2],
        ).wait()

        @pl.when(w < _EXTRA)
        def _():
            pltpu.make_async_copy(
                tab_hbm.at[:, pl.ds(0, _REM0 - _REM1)],
                cbufs[_NFULL % 2].at[:, pl.ds(0, _REM0 - _REM1)],
                csems[_NFULL % 2],
            ).wait()

        lo_r = chunk_lo(_NFULL)
        m_total = scan_matches(lo_r, lo_r + rem)
        emit_rows(m_total, cbufs[_NFULL % 2])

        @pl.when(w == _NW - 1)
        def _():
            pltpu.sync_copy(tail_hbm, tail_v)
            m_tail = scan_matches(jnp.int32(_FULL), jnp.int32(_VOCAB))

            def emit_t(j, _):
                row = jnp.full((16,), _scalar(mcol, j), jnp.int32)
                b = _scalar(mb, j)
                g0 = plsc.load_gather(tail_v, [row, iota])
                g1 = plsc.load_gather(tail_v, [row, iota + 16])
                stage[pl.ds(j * _DIM, 16)] = g0
                stage[pl.ds(j * _DIM + 16, 16)] = g1
                pltpu.async_copy(
                    stage.at[pl.ds(j * _DIM, _DIM)],
                    out_hbm.at[pl.ds(b * _DIM, _DIM)],
                    rsem,
                )
                return 0
            lax.fori_loop(0, m_tail, emit_t, 0)

            def drain_t(j, _):
                pltpu.make_async_copy(
                    out_hbm.at[pl.ds(0, _DIM)],
                    stage.at[pl.ds(0, _DIM)],
                    rsem,
                ).wait()
                return 0
            lax.fori_loop(0, m_tail, drain_t, 0)

    return k(indices, tab_t, tab_tail)


def kernel(indices, table):
    out_flat = _lookup(
        indices.astype(jnp.int32), table.T, table[_FULL:, :]
    )
    return out_flat.reshape(_BATCH, _DIM)


# speedup vs baseline: 1.0974x; 1.0974x over previous
import functools

import jax
import jax.numpy as jnp
from jax import lax
from jax.experimental import pallas as pl
from jax.experimental.pallas import tpu as pltpu
from jax.experimental.pallas import tpu_sc as plsc

_BATCH = 16384
_DIM = 32
_VOCAB = 1000000
_FULL = 999936
_TAIL = _VOCAB - _FULL
_NB = 4

_info = plsc.get_sparse_core_info()
_NC, _NS = _info.num_cores, _info.num_subcores
_NW = _NC * _NS

_LT = 128
_NLT = _FULL // _LT
_BASE_LT = _NLT // _NW
_EXTRA = _NLT - _BASE_LT * _NW

_CL = 3072
_NFC = (_BASE_LT * _LT) // _CL
_REM1 = _BASE_LT * _LT - _NFC * _CL
_REM0 = _REM1 + _LT
_NRANGE = _NFC + 1
_NBUF = 4

_LIST_CAP = 1024
_MATCH_CAP = 128
_SENT = 1 << 30


def _scalar(ref, j):
    return ref[pl.ds(j, 16)][0]


def _lookup(indices, tab3, tab_tail):
    mesh = plsc.VectorSubcoreMesh(core_axis_name="c", subcore_axis_name="s")

    @functools.partial(
        pl.kernel,
        mesh=mesh,
        compiler_params=pltpu.CompilerParams(needs_layout_passes=False),
        out_type=jax.ShapeDtypeStruct((_BATCH * _DIM,), jnp.float32),
        scratch_types=[
            pltpu.VMEM((_BATCH,), jnp.int32),
            pltpu.VMEM((_LIST_CAP + 16,), jnp.int32),
            pltpu.VMEM((_LIST_CAP + 16,), jnp.int32),
            pltpu.VMEM((_MATCH_CAP + 16,), jnp.int32),
            pltpu.VMEM((_MATCH_CAP + 16,), jnp.int32),
            pltpu.VMEM((8, _CL), jnp.float32),
            pltpu.VMEM((8, _CL), jnp.float32),
            pltpu.VMEM((8, _CL), jnp.float32),
            pltpu.VMEM((8, _CL), jnp.float32),
            pltpu.VMEM((_TAIL, _DIM), jnp.float32),
            pltpu.VMEM((_MATCH_CAP * _DIM,), jnp.float32),
            pltpu.SemaphoreType.DMA,
            pltpu.SemaphoreType.DMA,
            pltpu.SemaphoreType.DMA,
            pltpu.SemaphoreType.DMA,
            pltpu.SemaphoreType.DMA,
        ],
    )
    def k(idx_hbm, tab_hbm, tail_hbm, out_hbm, idx_all, vals, bpos, mcol, mb,
          cbuf0, cbuf1, cbuf2, cbuf3, tail_v, stage,
          csem0, csem1, csem2, csem3, rsem):
        w = lax.axis_index("s") * _NC + lax.axis_index("c")
        lo_lane = _LT * (_BASE_LT * w + jnp.minimum(w, _EXTRA))
        n_lt = _BASE_LT + jnp.where(w < _EXTRA, 1, 0)
        hi = jnp.where(w == _NW - 1, _VOCAB, lo_lane + n_lt * _LT)

        cbufs = (cbuf0, cbuf1, cbuf2, cbuf3)
        csems = (csem0, csem1, csem2, csem3)
        iota = lax.iota(jnp.int32, 16)

        def range_lo(c):
            return pl.multiple_of(lo_lane + c * _CL, _LT)

        def start_chunk(q):
            c, band = q // _NB, q % _NB
            buf, sem = cbufs[q % _NBUF], csems[q % _NBUF]
            if c < _NFC:
                pltpu.async_copy(
                    tab_hbm.at[band, :, pl.ds(range_lo(c), _CL)],
                    buf.at[:, pl.ds(0, _CL)], sem)
            else:
                @pl.when(w < _EXTRA)
                def _():
                    pltpu.async_copy(
                        tab_hbm.at[band, :, pl.ds(range_lo(c), _REM0)],
                        buf.at[:, pl.ds(0, _REM0)], sem)

                @pl.when(w >= _EXTRA)
                def _():
                    pltpu.async_copy(
                        tab_hbm.at[band, :, pl.ds(range_lo(c), _REM1)],
                        buf.at[:, pl.ds(0, _REM1)], sem)

        def wait_chunk(q):
            c = q // _NB
            buf, sem = cbufs[q % _NBUF], csems[q % _NBUF]
            if c < _NFC:
                pltpu.make_async_copy(
                    tab_hbm.at[0, :, pl.ds(0, _CL)],
                    buf.at[:, pl.ds(0, _CL)], sem).wait()
            else:
                pltpu.make_async_copy(
                    tab_hbm.at[0, :, pl.ds(0, _REM1)],
                    buf.at[:, pl.ds(0, _REM1)], sem).wait()

                @pl.when(w < _EXTRA)
                def _():
                    pltpu.make_async_copy(
                        tab_hbm.at[0, :, pl.ds(0, _REM0 - _REM1)],
                        buf.at[:, pl.ds(0, _REM0 - _REM1)], sem).wait()

        for q in range(_NBUF):
            start_chunk(q)

        pltpu.sync_copy(idx_hbm, idx_all)

        def filt(t, off):
            for u in range(4):
                vec = idx_all[pl.ds((t * 4 + u) * 16, 16)]
                m = (vec >= lo_lane) & (vec < hi)
                cnt = plsc.all_reduce_population_count(m)[0]
                off = jnp.minimum(off, _LIST_CAP - 16)
                plsc.store_compressed(vals.at[pl.ds(off, 16)], vec, mask=m)
                b = iota + (t * 4 + u) * 16
                plsc.store_compressed(bpos.at[pl.ds(off, 16)], b, mask=m)
                off = off + cnt
            return off

        total = lax.fori_loop(0, _BATCH // 64, filt, jnp.int32(0))
        vals[pl.ds(total, 16)] = jnp.full((16,), _SENT, jnp.int32)
        n_vec = (total + 15) // 16

        def scan_matches(lo_c, hi_c):
            def scan(t, off2):
                vec = vals[pl.ds(t * 16, 16)]
                m = (vec >= lo_c) & (vec < hi_c)
                cnt = plsc.all_reduce_population_count(m)[0]
                off2 = jnp.minimum(off2, _MATCH_CAP - 16)
                plsc.store_compressed(mcol.at[pl.ds(off2, 16)], vec - lo_c,
                                      mask=m)
                bvec = bpos[pl.ds(t * 16, 16)]
                plsc.store_compressed(mb.at[pl.ds(off2, 16)], bvec, mask=m)
                return off2 + cnt
            return lax.fori_loop(0, n_vec, scan, jnp.int32(0))

        def emit_band(m_total, buf, band):
            def emit16(g, _):
                colv = jnp.clip(mcol[pl.ds(g * 16, 16)], 0, _CL - 1)
                sidx = iota * _DIM + g * (16 * _DIM) + band * 8

                def per_dim(d, _):
                    dv = jnp.full((16,), d, jnp.int32)
                    gd = plsc.load_gather(buf, [dv, colv])
                    plsc.store_scatter(stage, [sidx + d], gd)
                    return 0

                lax.fori_loop(0, 8, per_dim, 0)
                return 0
            lax.fori_loop(0, (m_total + 15) // 16, emit16, 0)

        def send_rows(m_total):
            def send(j, _):
                b = _scalar(mb, j)
                pltpu.async_copy(
                    stage.at[pl.ds(j * _DIM, _DIM)],
                    out_hbm.at[pl.ds(b * _DIM, _DIM)],
                    rsem,
                )
                return 0
            lax.fori_loop(0, m_total, send, 0)

            def drain(j, _):
                pltpu.make_async_copy(
                    out_hbm.at[pl.ds(0, _DIM)],
                    stage.at[pl.ds(0, _DIM)],
                    rsem,
                ).wait()
                return 0
            lax.fori_loop(0, m_total, drain, 0)

        for c in range(_NRANGE):
            lo_c = range_lo(c)
            if c < _NFC:
                hi_c = lo_c + _CL
            else:
                hi_c = lo_c + jnp.where(w < _EXTRA, _REM0, _REM1)
            m_total = scan_matches(lo_c, hi_c)
            for band in range(_NB):
                q = c * _NB + band
                wait_chunk(q)
                emit_band(m_total, cbufs[q % _NBUF], band)
                if q + _NBUF < _NRANGE * _NB:
                    start_chunk(q + _NBUF)
            send_rows(m_total)

        @pl.when(w == _NW - 1)
        def _():
            pltpu.sync_copy(tail_hbm, tail_v)
            m_tail = scan_matches(jnp.int32(_FULL), jnp.int32(_VOCAB))

            def emit_t(j, _):
                row = jnp.full((16,), _scalar(mcol, j), jnp.int32)
                b = _scalar(mb, j)
                g0 = plsc.load_gather(tail_v, [row, iota])
                g1 = plsc.load_gather(tail_v, [row, iota + 16])
                stage[pl.ds(j * _DIM, 16)] = g0
                stage[pl.ds(j * _DIM + 16, 16)] = g1
                pltpu.async_copy(
                    stage.at[pl.ds(j * _DIM, _DIM)],
                    out_hbm.at[pl.ds(b * _DIM, _DIM)],
                    rsem,
                )
                return 0
            lax.fori_loop(0, m_tail, emit_t, 0)

            def drain_t(j, _):
                pltpu.make_async_copy(
                    out_hbm.at[pl.ds(0, _DIM)],
                    stage.at[pl.ds(0, _DIM)],
                    rsem,
                ).wait()
                return 0
            lax.fori_loop(0, m_tail, drain_t, 0)

    return k(indices, tab3, tab_tail)


def kernel(indices, table):
    tab3 = table.T.reshape(_NB, 8, _VOCAB)
    out_flat = _lookup(indices.astype(jnp.int32), tab3, table[_FULL:, :])
    return out_flat.reshape(_BATCH, _DIM)
